# baseline (speedup 1.0000x reference)
import functools

import jax
import jax.numpy as jnp
from jax import lax
from jax.experimental import pallas as pl
from jax.experimental.pallas import tpu as pltpu
from jax.experimental.pallas import tpu_sc as plsc

DIM = 64
BATCH = 16384
NC = 2
NS = 16
NW = NC * NS
BPW = BATCH // NW
CHUNK = 128
NCHUNK = BPW // CHUNK


def _sc_gather_body(x0, x1, x2, lut0, lut1, lut2, e0, e1, e2, idx0, idx1,
                    idx2, rows0, rows1, rows2, sem):
    wid = lax.axis_index("s") * NC + lax.axis_index("c")
    base = wid * BPW
    for src, idx in ((x0, idx0), (x1, idx1), (x2, idx2)):
        pltpu.sync_copy(src.at[pl.ds(base, BPW)], idx)
    copies = []
    for idx, lut, rows in ((idx0, lut0, rows0), (idx1, lut1, rows1),
                           (idx2, lut2, rows2)):
        for j in range(NCHUNK):
            copies.append(pltpu.async_copy(
                lut.at[idx.at[pl.ds(j * CHUNK, CHUNK)]],
                rows.at[pl.ds(j * CHUNK, CHUNK)],
                sem,
            ))
    for c in copies:
        c.wait()
    pltpu.sync_copy(rows0, e0.at[pl.ds(base, BPW)])
    pltpu.sync_copy(rows1, e1.at[pl.ds(base, BPW)])
    pltpu.sync_copy(rows2, e2.at[pl.ds(base, BPW)])


_EMB = jax.ShapeDtypeStruct((BATCH, DIM), jnp.float32)


@functools.cache
def _sc_gather():
    return pl.kernel(
        _sc_gather_body,
        mesh=plsc.VectorSubcoreMesh(core_axis_name="c", subcore_axis_name="s"),
        out_type=(_EMB, _EMB, _EMB),
        scratch_types=[
            pltpu.VMEM((BPW,), jnp.int32),
            pltpu.VMEM((BPW,), jnp.int32),
            pltpu.VMEM((BPW,), jnp.int32),
            pltpu.VMEM((BPW, DIM), jnp.float32),
            pltpu.VMEM((BPW, DIM), jnp.float32),
            pltpu.VMEM((BPW, DIM), jnp.float32),
            pltpu.SemaphoreType.DMA,
        ],
        compiler_params=pltpu.CompilerParams(use_tc_tiling_on_sc=False),
    )


BM = 2048


def _tc_proj_body(e0, e1, e2, w, b, o):
    acc = jnp.dot(e0[...], w[0:DIM, :], preferred_element_type=jnp.float32)
    acc += jnp.dot(e1[...], w[DIM:2 * DIM, :],
                   preferred_element_type=jnp.float32)
    acc += jnp.dot(e2[...], w[2 * DIM:3 * DIM, :],
                   preferred_element_type=jnp.float32)
    o[...] = acc * 8.0 + b[...]


def _tc_project(e0, e1, e2, w, b2d):
    grid = (BATCH // BM,)
    blk = pl.BlockSpec((BM, DIM), lambda i: (i, 0))
    return pl.pallas_call(
        _tc_proj_body,
        grid=grid,
        in_specs=[
            blk, blk, blk,
            pl.BlockSpec((3 * DIM, DIM), lambda i: (0, 0)),
            pl.BlockSpec((1, DIM), lambda i: (0, 0)),
        ],
        out_specs=blk,
        out_shape=jax.ShapeDtypeStruct((BATCH, DIM), jnp.float32),
    )(e0, e1, e2, w, b2d)


def kernel(x, lut0, lut1, lut2, W, b):
    xi = x.astype(jnp.int32)
    e0, e1, e2 = _sc_gather()(xi[:, 0], xi[:, 1], xi[:, 2], lut0, lut1, lut2)
    return _tc_project(e0, e1, e2, W, b.reshape(1, DIM))


# speedup vs baseline: 1.1026x; 1.1026x over previous
import functools

import jax
import jax.numpy as jnp
from jax import lax
from jax.experimental import pallas as pl
from jax.experimental.pallas import tpu as pltpu
from jax.experimental.pallas import tpu_sc as plsc

DIM = 64
BATCH = 16384
NC = 2
NS = 16
NW = NC * NS
BPW = BATCH // NW
CHUNK = 128
NCHUNK = BPW // CHUNK


def _sc_gather_body(xT, lut0, lut1, lut2, e01, e2, idx0, idx1, idx2,
                    rows0, rows1, rows2, sem):
    wid = lax.axis_index("s") * NC + lax.axis_index("c")
    base = wid * BPW
    for t, idx in enumerate((idx0, idx1, idx2)):
        pltpu.sync_copy(xT.at[t, pl.ds(base, BPW)], idx)
    copies = []
    for idx, lut, rows in ((idx0, lut0, rows0), (idx1, lut1, rows1),
                           (idx2, lut2, rows2)):
        for j in range(NCHUNK):
            copies.append(pltpu.async_copy(
                lut.at[idx.at[pl.ds(j * CHUNK, CHUNK)]],
                rows.at[pl.ds(j * CHUNK, CHUNK)],
                sem,
            ))
    for c in copies:
        c.wait()
    pltpu.sync_copy(rows0, e01.at[pl.ds(base, BPW), pl.ds(0, DIM)])
    pltpu.sync_copy(rows1, e01.at[pl.ds(base, BPW), pl.ds(DIM, DIM)])
    pltpu.sync_copy(rows2, e2.at[pl.ds(base, BPW), pl.ds(0, DIM)])


@functools.cache
def _sc_gather():
    packed = jax.ShapeDtypeStruct((BATCH, 2 * DIM), jnp.float32)
    return pl.kernel(
        _sc_gather_body,
        mesh=plsc.VectorSubcoreMesh(core_axis_name="c", subcore_axis_name="s"),
        out_type=(packed, packed),
        scratch_types=[
            pltpu.VMEM((BPW,), jnp.int32),
            pltpu.VMEM((BPW,), jnp.int32),
            pltpu.VMEM((BPW,), jnp.int32),
            pltpu.VMEM((BPW, DIM), jnp.float32),
            pltpu.VMEM((BPW, DIM), jnp.float32),
            pltpu.VMEM((BPW, DIM), jnp.float32),
            pltpu.SemaphoreType.DMA,
        ],
        compiler_params=pltpu.CompilerParams(use_tc_tiling_on_sc=False),
    )


BM = 2048


def _tc_proj_body(e01, e2, w, b, o):
    acc = jnp.dot(e01[...], w[0:2 * DIM, :],
                  preferred_element_type=jnp.float32)
    acc += jnp.dot(e2[:, 0:DIM], w[2 * DIM:3 * DIM, :],
                   preferred_element_type=jnp.float32)
    o[...] = acc * 8.0 + b[...]


def _tc_project(e01, e2, w, b2d):
    grid = (BATCH // BM,)
    eblk = pl.BlockSpec((BM, 2 * DIM), lambda i: (i, 0))
    return pl.pallas_call(
        _tc_proj_body,
        grid=grid,
        in_specs=[
            eblk, eblk,
            pl.BlockSpec((3 * DIM, DIM), lambda i: (0, 0)),
            pl.BlockSpec((1, DIM), lambda i: (0, 0)),
        ],
        out_specs=pl.BlockSpec((BM, DIM), lambda i: (i, 0)),
        out_shape=jax.ShapeDtypeStruct((BATCH, DIM), jnp.float32),
    )(e01, e2, w, b2d)


def kernel(x, lut0, lut1, lut2, W, b):
    xT = x.astype(jnp.int32).T
    e01, e2 = _sc_gather()(xT, lut0, lut1, lut2)
    return _tc_project(e01, e2, W, b.reshape(1, DIM))


# speedup vs baseline: 1.3721x; 1.2445x over previous
import functools

import jax
import jax.numpy as jnp
from jax import lax
from jax.experimental import pallas as pl
from jax.experimental.pallas import tpu as pltpu
from jax.experimental.pallas import tpu_sc as plsc

VOCAB = 100000
DIM = 64
BATCH = 16384
NC = 2
NS = 16
NW = NC * NS
BPW = BATCH // NW
CHUNK = 128
NCHUNK = BPW // CHUNK


def _sc_gather_body(xT, lut0, lut1, lut2, e01, e2, idx0, idx1, idx2,
                    rows0, rows1, rows2, sem):
    wid = lax.axis_index("s") * NC + lax.axis_index("c")
    base = wid * BPW
    for t, idx in enumerate((idx0, idx1, idx2)):
        pltpu.sync_copy(xT.at[t, pl.ds(base, BPW)], idx)
    copies = []
    for idx, lut, rows in ((idx0, lut0, rows0), (idx1, lut1, rows1),
                           (idx2, lut2, rows2)):
        for j in range(NCHUNK):
            copies.append(pltpu.async_copy(
                lut.at[idx.at[pl.ds(j * CHUNK, CHUNK)]],
                rows.at[pl.ds(j * CHUNK, CHUNK)],
                sem,
            ))
    for c in copies:
        c.wait()
    pltpu.sync_copy(rows0, e01.at[pl.ds(base, BPW), pl.ds(0, DIM)])
    pltpu.sync_copy(rows1, e01.at[pl.ds(base, BPW), pl.ds(DIM, DIM)])
    pltpu.sync_copy(rows2, e2.at[pl.ds(base, BPW), pl.ds(0, DIM)])


@functools.cache
def _sc_gather():
    packed = jax.ShapeDtypeStruct((BATCH, 2 * DIM), jnp.float32)
    return pl.kernel(
        _sc_gather_body,
        mesh=plsc.VectorSubcoreMesh(core_axis_name="c", subcore_axis_name="s"),
        out_type=(packed, packed),
        scratch_types=[
            pltpu.VMEM((BPW,), jnp.int32),
            pltpu.VMEM((BPW,), jnp.int32),
            pltpu.VMEM((BPW,), jnp.int32),
            pltpu.VMEM((BPW, DIM), jnp.float32),
            pltpu.VMEM((BPW, DIM), jnp.float32),
            pltpu.VMEM((BPW, DIM), jnp.float32),
            pltpu.SemaphoreType.DMA,
        ],
        compiler_params=pltpu.CompilerParams(use_tc_tiling_on_sc=False),
    )


CB = 2048
RB = CB // 2
VROWS = VOCAB // 2
RGRID = (VOCAB + CB - 1) // CB


def _tc_repack_body(l0, l1, l2, o0, o1, o2):
    for lr, orf in ((l0, o0), (l1, o1), (l2, o2)):
        t = jnp.transpose(lr[...])
        t3 = t.reshape(RB, 2, DIM)
        orf[...] = jnp.concatenate([t3[:, 0, :], t3[:, 1, :]], axis=1)


def _tc_repack(lt0, lt1, lt2):
    inblk = pl.BlockSpec((DIM, CB), lambda i: (0, i))
    outblk = pl.BlockSpec((RB, 2 * DIM), lambda i: (i, 0))
    oshape = jax.ShapeDtypeStruct((VROWS, 2 * DIM), jnp.float32)
    return pl.pallas_call(
        _tc_repack_body,
        grid=(RGRID,),
        in_specs=[inblk, inblk, inblk],
        out_specs=[outblk, outblk, outblk],
        out_shape=(oshape, oshape, oshape),
    )(lt0, lt1, lt2)


BM = 2048


def _tc_proj_body(e01, e2, w, b, o):
    acc = jnp.dot(e01[...], w[0:2 * DIM, :],
                  preferred_element_type=jnp.float32)
    acc += jnp.dot(e2[:, 0:DIM], w[2 * DIM:3 * DIM, :],
                   preferred_element_type=jnp.float32)
    o[...] = acc * 8.0 + b[...]


def _tc_project(e01, e2, w, b2d):
    grid = (BATCH // BM,)
    eblk = pl.BlockSpec((BM, 2 * DIM), lambda i: (i, 0))
    return pl.pallas_call(
        _tc_proj_body,
        grid=grid,
        in_specs=[
            eblk, eblk,
            pl.BlockSpec((3 * DIM, DIM), lambda i: (0, 0)),
            pl.BlockSpec((1, DIM), lambda i: (0, 0)),
        ],
        out_specs=pl.BlockSpec((BM, DIM), lambda i: (i, 0)),
        out_shape=jax.ShapeDtypeStruct((BATCH, DIM), jnp.float32),
    )(e01, e2, w, b2d)


def kernel(x, lut0, lut1, lut2, W, b):
    xT = x.astype(jnp.int32).T
    o0, o1, o2 = _tc_repack(lut0.T, lut1.T, lut2.T)
    g0 = o0.reshape(VOCAB, DIM)
    g1 = o1.reshape(VOCAB, DIM)
    g2 = o2.reshape(VOCAB, DIM)
    e01, e2 = _sc_gather()(xT, g0, g1, g2)
    return _tc_project(e01, e2, W, b.reshape(1, DIM))


# speedup vs baseline: 1.7818x; 1.2986x over previous
import functools

import jax
import jax.numpy as jnp
from jax import lax
from jax.experimental import pallas as pl
from jax.experimental.pallas import tpu as pltpu
from jax.experimental.pallas import tpu_sc as plsc

VOCAB = 100000
DIM = 64
BATCH = 16384
NC = 2
NS = 16
NW = NC * NS
BPW = BATCH // NW
CHUNK = 128
NCHUNK = BPW // CHUNK

CB = 2048
RGRID = (VOCAB + CB - 1) // CB


def _tc_repack_body(l0, l1, l2, o01, o22):
    o01[...] = jnp.transpose(jnp.concatenate([l0[...], l1[...]], axis=0))
    o22[...] = jnp.transpose(jnp.concatenate([l2[...], l2[...]], axis=0))


def _tc_repack(lt0, lt1, lt2):
    inblk = pl.BlockSpec((DIM, CB), lambda i: (0, i))
    outblk = pl.BlockSpec((CB, 2 * DIM), lambda i: (i, 0))
    oshape = jax.ShapeDtypeStruct((VOCAB, 2 * DIM), jnp.float32)
    return pl.pallas_call(
        _tc_repack_body,
        grid=(RGRID,),
        in_specs=[inblk, inblk, inblk],
        out_specs=[outblk, outblk],
        out_shape=(oshape, oshape),
    )(lt0, lt1, lt2)


def _sc_gather_body(xT, p01, p22, e01, e2, idx0, idx1, idx2, rows, sem):
    wid = lax.axis_index("s") * NC + lax.axis_index("c")
    base = wid * BPW
    for t, idx in enumerate((idx0, idx1, idx2)):
        pltpu.sync_copy(xT.at[t, pl.ds(base, BPW)], idx)
    for idx, src, dst, col in ((idx0, p01, e01, 0), (idx1, p01, e01, DIM),
                               (idx2, p22, e2, 0)):
        copies = []
        for j in range(NCHUNK):
            copies.append(pltpu.async_copy(
                src.at[idx.at[pl.ds(j * CHUNK, CHUNK)]],
                rows.at[pl.ds(j * CHUNK, CHUNK)],
                sem,
            ))
        for c in copies:
            c.wait()
        pltpu.sync_copy(rows.at[:, pl.ds(col, DIM)],
                        dst.at[pl.ds(base, BPW), pl.ds(col, DIM)])


@functools.cache
def _sc_gather():
    packed = jax.ShapeDtypeStruct((BATCH, 2 * DIM), jnp.float32)
    return pl.kernel(
        _sc_gather_body,
        mesh=plsc.VectorSubcoreMesh(core_axis_name="c", subcore_axis_name="s"),
        out_type=(packed, packed),
        scratch_types=[
            pltpu.VMEM((BPW,), jnp.int32),
            pltpu.VMEM((BPW,), jnp.int32),
            pltpu.VMEM((BPW,), jnp.int32),
            pltpu.VMEM((BPW, 2 * DIM), jnp.float32),
            pltpu.SemaphoreType.DMA,
        ],
        compiler_params=pltpu.CompilerParams(use_tc_tiling_on_sc=False),
    )


BM = 2048


def _tc_proj_body(e01, e2, w, b, o):
    acc = jnp.dot(e01[...], w[0:2 * DIM, :],
                  preferred_element_type=jnp.float32)
    acc += jnp.dot(e2[:, 0:DIM], w[2 * DIM:3 * DIM, :],
                   preferred_element_type=jnp.float32)
    o[...] = acc * 8.0 + b[...]


def _tc_project(e01, e2, w, b2d):
    grid = (BATCH // BM,)
    eblk = pl.BlockSpec((BM, 2 * DIM), lambda i: (i, 0))
    return pl.pallas_call(
        _tc_proj_body,
        grid=grid,
        in_specs=[
            eblk, eblk,
            pl.BlockSpec((3 * DIM, DIM), lambda i: (0, 0)),
            pl.BlockSpec((1, DIM), lambda i: (0, 0)),
        ],
        out_specs=pl.BlockSpec((BM, DIM), lambda i: (i, 0)),
        out_shape=jax.ShapeDtypeStruct((BATCH, DIM), jnp.float32),
    )(e01, e2, w, b2d)


def kernel(x, lut0, lut1, lut2, W, b):
    xT = x.astype(jnp.int32).T
    p01, p22 = _tc_repack(lut0.T, lut1.T, lut2.T)
    e01, e2 = _sc_gather()(xT, p01, p22)
    return _tc_project(e01, e2, W, b.reshape(1, DIM))


# speedup vs baseline: 1.9455x; 1.0919x over previous
import functools

import jax
import jax.numpy as jnp
from jax import lax
from jax.experimental import pallas as pl
from jax.experimental.pallas import tpu as pltpu
from jax.experimental.pallas import tpu_sc as plsc

VOCAB = 100000
DIM = 64
BATCH = 16384
NC = 2
NS = 16
NW = NC * NS
BPW = BATCH // NW
CHUNK = 128
NCHUNK = BPW // CHUNK
L = 16

CB = 2048
HB = CB // 2
RGRID = (VOCAB + CB - 1) // CB
P2ROWS = RGRID * HB


def _tc_repack_body(l0, l1, l2, o01, o2):
    o01[...] = jnp.transpose(jnp.concatenate([l0[...], l1[...]], axis=0))
    o2[...] = jnp.transpose(jnp.concatenate(
        [l2[:, 0:HB], l2[:, HB:CB]], axis=0))


def _tc_repack(lt0, lt1, lt2):
    inblk = pl.BlockSpec((DIM, CB), lambda i: (0, i))
    return pl.pallas_call(
        _tc_repack_body,
        grid=(RGRID,),
        in_specs=[inblk, inblk, inblk],
        out_specs=[pl.BlockSpec((CB, 2 * DIM), lambda i: (i, 0)),
                   pl.BlockSpec((HB, 2 * DIM), lambda i: (i, 0))],
        out_shape=(jax.ShapeDtypeStruct((VOCAB, 2 * DIM), jnp.float32),
                   jax.ShapeDtypeStruct((P2ROWS, 2 * DIM), jnp.float32)),
    )(lt0, lt1, lt2)


def _sc_gather_body(xT, p01, p2, e01, e2, idx0, idx1, idx2, bufA, bufB,
                    gsA, gsB, wsA, wsB):
    wid = lax.axis_index("s") * NC + lax.axis_index("c")
    base = wid * BPW
    for t, idx in enumerate((idx0, idx1, idx2)):
        pltpu.sync_copy(xT.at[t, pl.ds(base, BPW)], idx)
    for i in range(BPW // L):
        v = idx2[pl.ds(i * L, L)]
        idx2[pl.ds(i * L, L)] = ((v >> 11) << 10) | (v & 1023)

    jobs = []
    for idx, src, dst, col, full in ((idx0, p01, e01, 0, False),
                                     (idx1, p01, e01, DIM, False),
                                     (idx2, p2, e2, 0, True)):
        for j in range(NCHUNK):
            jobs.append((idx, src, dst, col, full, j))

    bufs, gs, ws = (bufA, bufB), (gsA, gsB), (wsA, wsB)
    gats = [None, None]
    wbs = [None, None]

    def issue_wb(kk):
        idx_, src_, dst_, col_, full_, j_ = jobs[kk]
        s_ = kk % 2
        gats[s_].wait()
        rows = pl.ds(base + j_ * CHUNK, CHUNK)
        if full_:
            wbs[s_] = pltpu.async_copy(bufs[s_], dst_.at[rows], ws[s_])
        else:
            wbs[s_] = pltpu.async_copy(
                bufs[s_].at[:, pl.ds(col_, DIM)],
                dst_.at[rows, pl.ds(col_, DIM)], ws[s_])

    for k, (idx, src, dst, col, full, j) in enumerate(jobs):
        s = k % 2
        if wbs[s] is not None:
            wbs[s].wait()
        gats[s] = pltpu.async_copy(
            src.at[idx.at[pl.ds(j * CHUNK, CHUNK)]], bufs[s], gs[s])
        if k >= 1:
            issue_wb(k - 1)
    issue_wb(len(jobs) - 1)
    wbs[0].wait()
    wbs[1].wait()


@functools.cache
def _sc_gather():
    packed = jax.ShapeDtypeStruct((BATCH, 2 * DIM), jnp.float32)
    return pl.kernel(
        _sc_gather_body,
        mesh=plsc.VectorSubcoreMesh(core_axis_name="c", subcore_axis_name="s"),
        out_type=(packed, packed),
        scratch_types=[
            pltpu.VMEM((BPW,), jnp.int32),
            pltpu.VMEM((BPW,), jnp.int32),
            pltpu.VMEM((BPW,), jnp.int32),
            pltpu.VMEM((CHUNK, 2 * DIM), jnp.float32),
            pltpu.VMEM((CHUNK, 2 * DIM), jnp.float32),
            pltpu.SemaphoreType.DMA,
            pltpu.SemaphoreType.DMA,
            pltpu.SemaphoreType.DMA,
            pltpu.SemaphoreType.DMA,
        ],
        compiler_params=pltpu.CompilerParams(use_tc_tiling_on_sc=False),
    )


BM = 2048


def _tc_proj_body(xb, e01, e2p, w, b, o):
    h = (xb[:, 2:3] >> 10) & 1
    e2 = jnp.where(h > 0, e2p[:, DIM:2 * DIM], e2p[:, 0:DIM])
    acc = lax.dot_general(w[0:2 * DIM, :], e01[...],
                          (((0,), (1,)), ((), ())),
                          preferred_element_type=jnp.float32)
    acc += lax.dot_general(w[2 * DIM:3 * DIM, :], e2,
                           (((0,), (1,)), ((), ())),
                           preferred_element_type=jnp.float32)
    o[...] = acc * 8.0 + b[...]


def _tc_project(x, e01, e2p, w, bcol):
    grid = (BATCH // BM,)
    eblk = pl.BlockSpec((BM, 2 * DIM), lambda i: (i, 0))
    outT = pl.pallas_call(
        _tc_proj_body,
        grid=grid,
        in_specs=[
            pl.BlockSpec((BM, 3), lambda i: (i, 0)),
            eblk, eblk,
            pl.BlockSpec((3 * DIM, DIM), lambda i: (0, 0)),
            pl.BlockSpec((DIM, 1), lambda i: (0, 0)),
        ],
        out_specs=pl.BlockSpec((DIM, BM), lambda i: (0, i)),
        out_shape=jax.ShapeDtypeStruct((DIM, BATCH), jnp.float32),
    )(x, e01, e2p, w, bcol)
    return outT.T


def kernel(x, lut0, lut1, lut2, W, b):
    xi = x.astype(jnp.int32)
    xT = xi.T
    p01, p2 = _tc_repack(lut0.T, lut1.T, lut2.T)
    e01, e2p = _sc_gather()(xT, p01, p2)
    return _tc_project(xi, e01, e2p, W, b.reshape(DIM, 1))
